# baseline (speedup 1.0000x reference)
import functools

import jax
import jax.numpy as jnp
from jax import lax
from jax.experimental import pallas as pl
from jax.experimental.pallas import tpu as pltpu
from jax.experimental.pallas import tpu_sc as plsc

CLS = 1_000_000
FEAT = 32
B = 16384
NW = 32
ROWS = B // NW
L = 16


def _body(xs_hbm, ys_hbm, center_hbm, out_hbm,
          idx2, hys, crows, xsv, cntf, zbuf, obuf, sem, histo):
    cid = lax.axis_index("c")
    sid = lax.axis_index("s")
    wid = sid * 2 + cid

    pltpu.sync_copy(ys_hbm.at[pl.ds(wid * 4, 4)], idx2)
    pltpu.sync_copy(ys_hbm.at[pl.ds(sid * 8, 8)], hys)

    copies = [
        pltpu.async_copy(center_hbm.at[idx2.at[j]],
                         crows.at[pl.ds(j * 128, 128)], sem)
        for j in range(4)
    ]
    copies.append(pltpu.async_copy(xs_hbm.at[pl.ds(wid * ROWS, ROWS)],
                                   xsv, sem))

    for i in range(8):
        zbuf[pl.ds(i * L, L)] = jnp.zeros((L,), jnp.int32)
    for j in range(8):
        pltpu.sync_copy(zbuf, histo.at[hys.at[j]])
    plsc.subcore_barrier()

    for i in range(8):
        zbuf[pl.ds(i * L, L)] = jnp.ones((L,), jnp.int32)
    for j in range(8):
        pltpu.sync_copy(zbuf, histo.at[hys.at[j]], add=True)
    plsc.subcore_barrier()

    for j in range(4):
        pltpu.sync_copy(histo.at[idx2.at[j]], cntf.at[pl.ds(j * 128, 128)])

    for c in copies:
        c.wait()

    def group(g, acc):
        ci = cntf[pl.ds(g * L, L)]
        w = 1.0 / ci.astype(jnp.float32)
        for r in range(L):
            row = g * L + r
            x0 = xsv[row, pl.ds(0, L)]
            x1 = xsv[row, pl.ds(L, L)]
            c0 = crows[row, pl.ds(0, L)]
            c1 = crows[row, pl.ds(L, L)]
            d0 = x0 - c0
            d1 = x1 - c1
            p = d0 * d0 + d1 * d1
            wr = lax.gather(
                w, jnp.full((L, 1), r, jnp.int32),
                dimension_numbers=lax.GatherDimensionNumbers(
                    offset_dims=(), collapsed_slice_dims=(0,),
                    start_index_map=(0,)),
                slice_sizes=(1,),
                mode=lax.GatherScatterMode.PROMISE_IN_BOUNDS)
            acc = acc + p * wr
        return acc

    acc = lax.fori_loop(0, ROWS // L, group, jnp.zeros((L,), jnp.float32))
    obuf[...] = acc
    pltpu.sync_copy(obuf, out_hbm.at[wid])


@jax.jit
def _center_loss(xs, ys2, center):
    kfn = pl.kernel(
        _body,
        out_type=jax.ShapeDtypeStruct((NW, L), jnp.float32),
        mesh=plsc.VectorSubcoreMesh(core_axis_name="c", subcore_axis_name="s",
                                    num_cores=2, num_subcores=16),
        compiler_params=pltpu.CompilerParams(use_tc_tiling_on_sc=False),
        scratch_types=[
            pltpu.VMEM((4, 128), jnp.int32),
            pltpu.VMEM((8, 128), jnp.int32),
            pltpu.VMEM((ROWS, FEAT), jnp.float32),
            pltpu.VMEM((ROWS, FEAT), jnp.float32),
            pltpu.VMEM((ROWS,), jnp.int32),
            pltpu.VMEM((128,), jnp.int32),
            pltpu.VMEM((L,), jnp.float32),
            pltpu.SemaphoreType.DMA,
            pltpu.VMEM_SHARED((CLS,), jnp.int32),
        ],
    )
    parts = kfn(xs, ys2, center)
    return jnp.sum(parts) * 0.5


def kernel(xs, ys, center):
    ys2 = ys.astype(jnp.int32).reshape(128, 128)
    return _center_loss(xs, ys2, center)
